# baseline (speedup 1.0000x reference)
import functools

import jax
import jax.numpy as jnp
from jax.experimental import pallas as pl
from jax.experimental.pallas import tpu as pltpu


def _pick_block(n: int, cap: int) -> int:
    best = 0
    for d in range(8, min(n, cap) + 1, 8):
        if n % d == 0:
            best = d
    return best if best > 0 else n


def _v_body(x_ref, w1_ref, v_ref, *, C: int, H: int):
    for k in range(C):
        v = jnp.dot(x_ref[k], w1_ref[...], preferred_element_type=jnp.float32)
        v_ref[:, k * H:(k + 1) * H] = v.astype(jnp.bfloat16)


def _issue_group(hg_ref, buf_ref, sem_ref, group, slot, P, ch):
    for p in range(P):
        pltpu.make_async_copy(
            hg_ref.at[pl.ds(group * (P * ch) + p * ch, ch), :],
            buf_ref.at[slot, pl.ds(p * ch, ch), :],
            sem_ref.at[slot, p],
        ).start()


def _wait_group(hg_ref, buf_ref, sem_ref, group, slot, P, ch):
    for p in range(P):
        pltpu.make_async_copy(
            hg_ref.at[pl.ds(group * (P * ch) + p * ch, ch), :],
            buf_ref.at[slot, pl.ds(p * ch, ch), :],
            sem_ref.at[slot, p],
        ).wait()


def _stream_hg(hg_ref, buf_ref, sem_ref, steps, P, ch, nslots=2):
    i = pl.program_id(0)
    slot = jax.lax.rem(i, nslots)

    @pl.when(i == 0)
    def _():
        for d in range(min(nslots - 1, steps)):
            _issue_group(hg_ref, buf_ref, sem_ref, d, d % nslots, P, ch)

    @pl.when(i + nslots - 1 < steps)
    def _():
        _issue_group(hg_ref, buf_ref, sem_ref, i + nslots - 1,
                     jax.lax.rem(i + nslots - 1, nslots), P, ch)

    _wait_group(hg_ref, buf_ref, sem_ref, i, slot, P, ch)
    return buf_ref[slot]


def _uni_body(hg_ref, v_ref, b1_ref, w2_ref, u_ref, buf_ref, sem_ref,
              *, steps, P, ch):
    i = pl.program_id(0)
    G = P * ch
    hgb = _stream_hg(hg_ref, buf_ref, sem_ref, steps, P, ch, nslots=3)
    acc = jnp.dot(hgb.astype(jnp.bfloat16), v_ref[...],
                  preferred_element_type=jnp.float32)
    vi = v_ref[pl.ds(i * G, G), :].astype(jnp.float32)
    h = acc + vi + b1_ref[...]
    h = jnp.maximum(h, 0.0)
    u_ref[...] = jnp.dot(h, w2_ref[...], preferred_element_type=jnp.float32)


def _out_body(hg_ref, u_ref, b2_ref, o_ref, buf_ref, sem_ref,
              *, steps, P, ch):
    i = pl.program_id(0)
    G = P * ch
    hgb = _stream_hg(hg_ref, buf_ref, sem_ref, steps, P, ch, nslots=3)
    acc = jnp.dot(hgb.astype(jnp.bfloat16), u_ref[...].astype(jnp.bfloat16),
                  preferred_element_type=jnp.float32)
    o_ref[...] = acc + u_ref[pl.ds(i * G, G), :] + b2_ref[...]


def kernel(x_list, hg, W1, b1, W2, b2):
    C, N, F = x_list.shape
    H = W1.shape[1]
    CH = C * H
    O = W2.shape[1]

    bma = _pick_block(N, 2048)
    ch = _pick_block(N, 80)
    P = min(5, N // ch)
    G = P * ch
    steps = N // G

    b1c = jnp.tile(b1, C).reshape(1, CH)
    b2r = b2.reshape(1, O)

    V = pl.pallas_call(
        functools.partial(_v_body, C=C, H=H),
        grid=(N // bma,),
        in_specs=[
            pl.BlockSpec((C, bma, F), lambda i: (0, i, 0)),
            pl.BlockSpec((F, H), lambda i: (0, 0)),
        ],
        out_specs=pl.BlockSpec((bma, CH), lambda i: (i, 0)),
        out_shape=jax.ShapeDtypeStruct((N, CH), jnp.bfloat16),
        compiler_params=pltpu.CompilerParams(
            dimension_semantics=("parallel",)),
    )(x_list, W1)

    hg_spec = pl.BlockSpec(memory_space=pltpu.MemorySpace.HBM)
    const1 = pl.Buffered(buffer_count=1)

    u = pl.pallas_call(
        functools.partial(_uni_body, steps=steps, P=P, ch=ch),
        grid=(steps,),
        in_specs=[
            hg_spec,
            pl.BlockSpec((N, CH), lambda i: (0, 0), pipeline_mode=const1),
            pl.BlockSpec((1, CH), lambda i: (0, 0), pipeline_mode=const1),
            pl.BlockSpec((CH, O), lambda i: (0, 0), pipeline_mode=const1),
        ],
        out_specs=pl.BlockSpec((G, O), lambda i: (i, 0)),
        out_shape=jax.ShapeDtypeStruct((N, O), jnp.float32),
        scratch_shapes=[
            pltpu.VMEM((3, G, N), jnp.float32),
            pltpu.SemaphoreType.DMA((3, P)),
        ],
        compiler_params=pltpu.CompilerParams(
            dimension_semantics=("arbitrary",),
            vmem_limit_bytes=67108864),
    )(hg, V, b1c, W2)

    out = pl.pallas_call(
        functools.partial(_out_body, steps=steps, P=P, ch=ch),
        grid=(steps,),
        in_specs=[
            hg_spec,
            pl.BlockSpec((N, O), lambda i: (0, 0), pipeline_mode=const1),
            pl.BlockSpec((1, O), lambda i: (0, 0), pipeline_mode=const1),
        ],
        out_specs=pl.BlockSpec((G, O), lambda i: (i, 0)),
        out_shape=jax.ShapeDtypeStruct((N, O), jnp.float32),
        scratch_shapes=[
            pltpu.VMEM((3, G, N), jnp.float32),
            pltpu.SemaphoreType.DMA((3, P)),
        ],
        compiler_params=pltpu.CompilerParams(
            dimension_semantics=("arbitrary",),
            vmem_limit_bytes=67108864),
    )(hg, u, b2r)

    return out


# speedup vs baseline: 1.0212x; 1.0212x over previous
import functools

import jax
import jax.numpy as jnp
from jax.experimental import pallas as pl
from jax.experimental.pallas import tpu as pltpu


def _pick_block(n: int, cap: int) -> int:
    best = 0
    for d in range(8, min(n, cap) + 1, 8):
        if n % d == 0:
            best = d
    return best if best > 0 else n


def _v_body(x_ref, w1_ref, v_ref, *, C: int, H: int):
    for k in range(C):
        v = jnp.dot(x_ref[k], w1_ref[...], preferred_element_type=jnp.float32)
        v_ref[:, k * H:(k + 1) * H] = v.astype(jnp.bfloat16)


def _issue_group(hg_ref, buf_ref, sem_ref, group, slot, P, ch):
    for p in range(P):
        pltpu.make_async_copy(
            hg_ref.at[pl.ds(group * (P * ch) + p * ch, ch), :],
            buf_ref.at[slot, pl.ds(p * ch, ch), :],
            sem_ref.at[slot, p],
        ).start()


def _wait_group(hg_ref, buf_ref, sem_ref, group, slot, P, ch):
    for p in range(P):
        pltpu.make_async_copy(
            hg_ref.at[pl.ds(group * (P * ch) + p * ch, ch), :],
            buf_ref.at[slot, pl.ds(p * ch, ch), :],
            sem_ref.at[slot, p],
        ).wait()


def _stream_hg(hg_ref, buf_ref, sem_ref, steps, P, ch, nslots=2):
    i = pl.program_id(0)
    slot = jax.lax.rem(i, nslots)

    @pl.when(i == 0)
    def _():
        for d in range(min(nslots - 1, steps)):
            _issue_group(hg_ref, buf_ref, sem_ref, d, d % nslots, P, ch)

    @pl.when(i + nslots - 1 < steps)
    def _():
        _issue_group(hg_ref, buf_ref, sem_ref, i + nslots - 1,
                     jax.lax.rem(i + nslots - 1, nslots), P, ch)

    _wait_group(hg_ref, buf_ref, sem_ref, i, slot, P, ch)
    return buf_ref[slot]


def _uni_body(hg_ref, v_ref, b1_ref, w2_ref, u_ref, buf_ref, sem_ref,
              *, steps, P, ch):
    i = pl.program_id(0)
    G = P * ch
    hgb = _stream_hg(hg_ref, buf_ref, sem_ref, steps, P, ch, nslots=3)
    acc = jnp.dot(hgb.astype(jnp.bfloat16), v_ref[...],
                  preferred_element_type=jnp.float32)
    vi = v_ref[pl.ds(i * G, G), :].astype(jnp.float32)
    h = acc + vi + b1_ref[...]
    h = jnp.maximum(h, 0.0)
    u_ref[...] = jnp.dot(h, w2_ref[...], preferred_element_type=jnp.float32)


def _out_body(hg_ref, u_ref, b2_ref, o_ref, buf_ref, sem_ref,
              *, steps, P, ch):
    i = pl.program_id(0)
    G = P * ch
    hgb = _stream_hg(hg_ref, buf_ref, sem_ref, steps, P, ch, nslots=3)
    acc = jnp.dot(hgb.astype(jnp.bfloat16), u_ref[...].astype(jnp.bfloat16),
                  preferred_element_type=jnp.float32)
    o_ref[...] = acc + u_ref[pl.ds(i * G, G), :] + b2_ref[...]


def kernel(x_list, hg, W1, b1, W2, b2):
    C, N, F = x_list.shape
    H = W1.shape[1]
    CH = C * H
    O = W2.shape[1]

    bma = _pick_block(N, 2048)
    ch = _pick_block(N, 40)
    P = min(10, N // ch)
    G = P * ch
    steps = N // G

    b1c = jnp.tile(b1, C).reshape(1, CH)
    b2r = b2.reshape(1, O)

    V = pl.pallas_call(
        functools.partial(_v_body, C=C, H=H),
        grid=(N // bma,),
        in_specs=[
            pl.BlockSpec((C, bma, F), lambda i: (0, i, 0)),
            pl.BlockSpec((F, H), lambda i: (0, 0)),
        ],
        out_specs=pl.BlockSpec((bma, CH), lambda i: (i, 0)),
        out_shape=jax.ShapeDtypeStruct((N, CH), jnp.bfloat16),
        compiler_params=pltpu.CompilerParams(
            dimension_semantics=("parallel",)),
    )(x_list, W1)

    hg_spec = pl.BlockSpec(memory_space=pltpu.MemorySpace.HBM)
    const1 = pl.Buffered(buffer_count=1)

    u = pl.pallas_call(
        functools.partial(_uni_body, steps=steps, P=P, ch=ch),
        grid=(steps,),
        in_specs=[
            hg_spec,
            pl.BlockSpec((N, CH), lambda i: (0, 0), pipeline_mode=const1),
            pl.BlockSpec((1, CH), lambda i: (0, 0), pipeline_mode=const1),
            pl.BlockSpec((CH, O), lambda i: (0, 0), pipeline_mode=const1),
        ],
        out_specs=pl.BlockSpec((G, O), lambda i: (i, 0)),
        out_shape=jax.ShapeDtypeStruct((N, O), jnp.float32),
        scratch_shapes=[
            pltpu.VMEM((3, G, N), jnp.float32),
            pltpu.SemaphoreType.DMA((3, P)),
        ],
        compiler_params=pltpu.CompilerParams(
            dimension_semantics=("arbitrary",),
            vmem_limit_bytes=67108864),
    )(hg, V, b1c, W2)

    out = pl.pallas_call(
        functools.partial(_out_body, steps=steps, P=P, ch=ch),
        grid=(steps,),
        in_specs=[
            hg_spec,
            pl.BlockSpec((N, O), lambda i: (0, 0), pipeline_mode=const1),
            pl.BlockSpec((1, O), lambda i: (0, 0), pipeline_mode=const1),
        ],
        out_specs=pl.BlockSpec((G, O), lambda i: (i, 0)),
        out_shape=jax.ShapeDtypeStruct((N, O), jnp.float32),
        scratch_shapes=[
            pltpu.VMEM((3, G, N), jnp.float32),
            pltpu.SemaphoreType.DMA((3, P)),
        ],
        compiler_params=pltpu.CompilerParams(
            dimension_semantics=("arbitrary",),
            vmem_limit_bytes=67108864),
    )(hg, u, b2r)

    return out
